# baseline (speedup 1.0000x reference)
import functools

import jax
import jax.numpy as jnp
from jax.experimental import pallas as pl
from jax.experimental.pallas import tpu as pltpu

DIM = 1024
HIDDEN = 2816
NUM_EXPERTS = 8
TOP_K = 2
TM = 128

_INTERPRET = False


def _router_kernel(x_ref, rw_ref, w_ref, e_ref):
    x = x_ref[...]
    rw = rw_ref[...]
    logits = jax.lax.dot_general(
        x, rw, (((1,), (1,)), ((), ())), preferred_element_type=jnp.float32)
    m = jnp.max(logits, axis=-1, keepdims=True)
    ex = jnp.exp(logits - m)
    probs = ex / jnp.sum(ex, axis=-1, keepdims=True)

    lane = jax.lax.broadcasted_iota(jnp.int32, probs.shape, 1)
    w1 = jnp.max(probs, axis=-1, keepdims=True)
    e1 = jnp.min(jnp.where(probs == w1, lane, NUM_EXPERTS), axis=-1,
                 keepdims=True)
    masked = jnp.where(lane == e1, -1.0, probs)
    w2 = jnp.max(masked, axis=-1, keepdims=True)
    e2 = jnp.min(jnp.where(masked == w2, lane, NUM_EXPERTS), axis=-1,
                 keepdims=True)
    s = w1 + w2
    w_ref[...] = jnp.concatenate([w1 / s, w2 / s], axis=-1)
    e_ref[...] = jnp.concatenate([e1, e2], axis=-1)


def _route(x, router_w):
    T = x.shape[0]
    return pl.pallas_call(
        _router_kernel,
        out_shape=(
            jax.ShapeDtypeStruct((T, TOP_K), jnp.float32),
            jax.ShapeDtypeStruct((T, TOP_K), jnp.int32),
        ),
        interpret=_INTERPRET,
    )(x, router_w)


NH = 2
HB = HIDDEN // NH


def _mlp_kernel(te_ref, xg_ref, gate_ref, up_ref, down_ref, out_ref):
    del te_ref
    x = xg_ref[...]
    g = jax.lax.dot_general(
        x, gate_ref[0].astype(jnp.bfloat16), (((1,), (1,)), ((), ())),
        preferred_element_type=jnp.float32)
    u = jax.lax.dot_general(
        x, up_ref[0].astype(jnp.bfloat16), (((1,), (1,)), ((), ())),
        preferred_element_type=jnp.float32)
    h = (g * jax.lax.logistic(g)) * u
    out_ref[...] = jax.lax.dot_general(
        h.astype(jnp.bfloat16), down_ref[0].astype(jnp.bfloat16),
        (((1,), (1,)), ((), ())),
        preferred_element_type=jnp.float32)


def _grouped_mlp_half(xg, tile_expert, gate_w, up_w, down_w, h_idx):
    R = xg.shape[0]
    n_tiles = R // TM
    grid_spec = pltpu.PrefetchScalarGridSpec(
        num_scalar_prefetch=1,
        grid=(n_tiles,),
        in_specs=[
            pl.BlockSpec((TM, DIM), lambda i, te: (i, 0)),
            pl.BlockSpec((1, HB, DIM), lambda i, te: (te[i], h_idx, 0)),
            pl.BlockSpec((1, HB, DIM), lambda i, te: (te[i], h_idx, 0)),
            pl.BlockSpec((1, DIM, HB), lambda i, te: (te[i], 0, h_idx)),
        ],
        out_specs=pl.BlockSpec((TM, DIM), lambda i, te: (i, 0)),
    )
    return pl.pallas_call(
        _mlp_kernel,
        grid_spec=grid_spec,
        out_shape=jax.ShapeDtypeStruct((R, DIM), jnp.float32),
        interpret=_INTERPRET,
    )(tile_expert, xg, gate_w, up_w, down_w)


def kernel(hidden_states, router_w, gate_w, up_w, down_w):
    B, S, D = hidden_states.shape
    T = B * S
    x = hidden_states.reshape(T, D)

    w, e = _route(x, router_w)

    e_flat = e.reshape(-1)
    oh = (e_flat[:, None] == jnp.arange(NUM_EXPERTS)[None, :]).astype(jnp.int32)
    pos = jnp.cumsum(oh, axis=0) - oh
    pos_i = jnp.sum(pos * oh, axis=-1)
    counts = jnp.sum(oh, axis=0)
    padded = ((counts + TM - 1) // TM) * TM
    ends = jnp.cumsum(padded)
    offs = ends - padded
    slot = offs[e_flat] + pos_i

    R = T * TOP_K + NUM_EXPERTS * TM
    gidx = jnp.zeros((R,), jnp.int32).at[slot].set(
        jnp.arange(T * TOP_K, dtype=jnp.int32) // TOP_K)
    n_tiles = R // TM
    tile_expert = jnp.minimum(
        jnp.sum(jnp.arange(n_tiles)[:, None] >= (ends // TM)[None, :], axis=-1),
        NUM_EXPERTS - 1).astype(jnp.int32)

    xg = jnp.take(x.astype(jnp.bfloat16), gidx, axis=0)
    y0 = _grouped_mlp_half(xg, tile_expert, gate_w, up_w, down_w, 0)
    y1 = _grouped_mlp_half(xg, tile_expert, gate_w, up_w, down_w, 1)

    s_tok = slot.reshape(T, TOP_K)
    y = y0 + y1
    out = (w[:, 0:1] * jnp.take(y, s_tok[:, 0], axis=0)
           + w[:, 1:2] * jnp.take(y, s_tok[:, 1], axis=0))
    return out.reshape(B, S, D)


# speedup vs baseline: 1.3839x; 1.3839x over previous
import functools

import jax
import jax.numpy as jnp
from jax.experimental import pallas as pl
from jax.experimental.pallas import tpu as pltpu

DIM = 1024
HIDDEN = 2816
NUM_EXPERTS = 8
TOP_K = 2
TM = 256

_INTERPRET = False


def _router_kernel(x_ref, rw_ref, w_ref, e_ref):
    x = x_ref[...]
    rw = rw_ref[...]
    logits = jax.lax.dot_general(
        x, rw, (((1,), (1,)), ((), ())), preferred_element_type=jnp.float32)
    m = jnp.max(logits, axis=-1, keepdims=True)
    ex = jnp.exp(logits - m)
    probs = ex / jnp.sum(ex, axis=-1, keepdims=True)

    lane = jax.lax.broadcasted_iota(jnp.int32, probs.shape, 1)
    w1 = jnp.max(probs, axis=-1, keepdims=True)
    e1 = jnp.min(jnp.where(probs == w1, lane, NUM_EXPERTS), axis=-1,
                 keepdims=True)
    masked = jnp.where(lane == e1, -1.0, probs)
    w2 = jnp.max(masked, axis=-1, keepdims=True)
    e2 = jnp.min(jnp.where(masked == w2, lane, NUM_EXPERTS), axis=-1,
                 keepdims=True)
    s = w1 + w2
    w_ref[...] = jnp.concatenate([w1 / s, w2 / s], axis=-1)
    e_ref[...] = jnp.concatenate([e1, e2], axis=-1)


def _route(x, router_w):
    T = x.shape[0]
    return pl.pallas_call(
        _router_kernel,
        out_shape=(
            jax.ShapeDtypeStruct((T, TOP_K), jnp.float32),
            jax.ShapeDtypeStruct((T, TOP_K), jnp.int32),
        ),
        interpret=_INTERPRET,
    )(x, router_w)


NH = 2
HB = HIDDEN // NH


def _mlp_kernel(te_ref, xg_ref, gate_ref, up_ref, down_ref, out_ref):
    del te_ref
    x = xg_ref[...]
    g = jax.lax.dot_general(
        x, gate_ref[0].astype(jnp.bfloat16), (((1,), (1,)), ((), ())),
        preferred_element_type=jnp.float32)
    u = jax.lax.dot_general(
        x, up_ref[0].astype(jnp.bfloat16), (((1,), (1,)), ((), ())),
        preferred_element_type=jnp.float32)
    h = (g * jax.lax.logistic(g)) * u
    out_ref[...] = jax.lax.dot_general(
        h.astype(jnp.bfloat16), down_ref[0].astype(jnp.bfloat16),
        (((1,), (1,)), ((), ())),
        preferred_element_type=jnp.float32)


def _grouped_mlp_half(xg, tile_expert, gate_w, up_w, down_w, h_idx):
    R = xg.shape[0]
    n_tiles = R // TM
    grid_spec = pltpu.PrefetchScalarGridSpec(
        num_scalar_prefetch=1,
        grid=(n_tiles,),
        in_specs=[
            pl.BlockSpec((TM, DIM), lambda i, te: (i, 0)),
            pl.BlockSpec((1, HB, DIM), lambda i, te: (te[i], h_idx, 0)),
            pl.BlockSpec((1, HB, DIM), lambda i, te: (te[i], h_idx, 0)),
            pl.BlockSpec((1, DIM, HB), lambda i, te: (te[i], 0, h_idx)),
        ],
        out_specs=pl.BlockSpec((TM, DIM), lambda i, te: (i, 0)),
    )
    return pl.pallas_call(
        _mlp_kernel,
        grid_spec=grid_spec,
        out_shape=jax.ShapeDtypeStruct((R, DIM), jnp.float32),
        interpret=_INTERPRET,
    )(tile_expert, xg, gate_w, up_w, down_w)


def kernel(hidden_states, router_w, gate_w, up_w, down_w):
    B, S, D = hidden_states.shape
    T = B * S
    x = hidden_states.reshape(T, D)

    w, e = _route(x, router_w)

    e_flat = e.reshape(-1)
    oh = (e_flat[:, None] == jnp.arange(NUM_EXPERTS)[None, :]).astype(jnp.int32)
    pos = jnp.cumsum(oh, axis=0) - oh
    pos_i = jnp.sum(pos * oh, axis=-1)
    counts = jnp.sum(oh, axis=0)
    padded = ((counts + TM - 1) // TM) * TM
    ends = jnp.cumsum(padded)
    offs = ends - padded
    slot = offs[e_flat] + pos_i

    R = T * TOP_K + NUM_EXPERTS * TM
    gidx = jnp.zeros((R,), jnp.int32).at[slot].set(
        jnp.arange(T * TOP_K, dtype=jnp.int32) // TOP_K)
    n_tiles = R // TM
    tile_expert = jnp.minimum(
        jnp.sum(jnp.arange(n_tiles)[:, None] >= (ends // TM)[None, :], axis=-1),
        NUM_EXPERTS - 1).astype(jnp.int32)

    xg = jnp.take(x.astype(jnp.bfloat16), gidx, axis=0)
    y0 = _grouped_mlp_half(xg, tile_expert, gate_w, up_w, down_w, 0)
    y1 = _grouped_mlp_half(xg, tile_expert, gate_w, up_w, down_w, 1)

    s_tok = slot.reshape(T, TOP_K)
    y = y0 + y1
    out = (w[:, 0:1] * jnp.take(y, s_tok[:, 0], axis=0)
           + w[:, 1:2] * jnp.take(y, s_tok[:, 1], axis=0))
    return out.reshape(B, S, D)


# speedup vs baseline: 1.4295x; 1.0330x over previous
import functools

import jax
import jax.numpy as jnp
from jax.experimental import pallas as pl
from jax.experimental.pallas import tpu as pltpu

DIM = 1024
HIDDEN = 2816
NUM_EXPERTS = 8
TOP_K = 2
TM = 256

_INTERPRET = False


def _router_kernel(x_ref, rw_ref, w_ref, e_ref):
    x = x_ref[...]
    rw = rw_ref[...]
    logits = jax.lax.dot_general(
        x, rw, (((1,), (1,)), ((), ())), preferred_element_type=jnp.float32)
    m = jnp.max(logits, axis=-1, keepdims=True)
    ex = jnp.exp(logits - m)
    probs = ex / jnp.sum(ex, axis=-1, keepdims=True)

    lane = jax.lax.broadcasted_iota(jnp.int32, probs.shape, 1)
    w1 = jnp.max(probs, axis=-1, keepdims=True)
    e1 = jnp.min(jnp.where(probs == w1, lane, NUM_EXPERTS), axis=-1,
                 keepdims=True)
    masked = jnp.where(lane == e1, -1.0, probs)
    w2 = jnp.max(masked, axis=-1, keepdims=True)
    e2 = jnp.min(jnp.where(masked == w2, lane, NUM_EXPERTS), axis=-1,
                 keepdims=True)
    s = w1 + w2
    w_ref[...] = jnp.concatenate([w1 / s, w2 / s], axis=-1)
    e_ref[...] = jnp.concatenate([e1, e2], axis=-1)


def _route(x, router_w):
    T = x.shape[0]
    return pl.pallas_call(
        _router_kernel,
        out_shape=(
            jax.ShapeDtypeStruct((T, TOP_K), jnp.float32),
            jax.ShapeDtypeStruct((T, TOP_K), jnp.int32),
        ),
        interpret=_INTERPRET,
    )(x, router_w)


NH = 2
HB = HIDDEN // NH


def _mlp_kernel(te_ref, xg_ref, gate_ref, up_ref, down_ref, out_ref):
    del te_ref
    x = xg_ref[...]
    g = jax.lax.dot_general(
        x, gate_ref[0].astype(jnp.bfloat16), (((1,), (1,)), ((), ())),
        preferred_element_type=jnp.float32)
    u = jax.lax.dot_general(
        x, up_ref[0].astype(jnp.bfloat16), (((1,), (1,)), ((), ())),
        preferred_element_type=jnp.float32)
    h = (g * jax.lax.logistic(g)) * u
    out_ref[...] = jax.lax.dot_general(
        h.astype(jnp.bfloat16), down_ref[0].astype(jnp.bfloat16),
        (((1,), (1,)), ((), ())),
        preferred_element_type=jnp.float32)


def _grouped_mlp_half(xg, tile_expert, gate_w, up_w, down_w, h_idx):
    R = xg.shape[0]
    n_tiles = R // TM
    grid_spec = pltpu.PrefetchScalarGridSpec(
        num_scalar_prefetch=1,
        grid=(n_tiles,),
        in_specs=[
            pl.BlockSpec((TM, DIM), lambda i, te: (i, 0)),
            pl.BlockSpec((1, HB, DIM), lambda i, te: (te[i], h_idx, 0)),
            pl.BlockSpec((1, HB, DIM), lambda i, te: (te[i], h_idx, 0)),
            pl.BlockSpec((1, DIM, HB), lambda i, te: (te[i], 0, h_idx)),
        ],
        out_specs=pl.BlockSpec((TM, DIM), lambda i, te: (i, 0)),
    )
    return pl.pallas_call(
        _mlp_kernel,
        grid_spec=grid_spec,
        out_shape=jax.ShapeDtypeStruct((R, DIM), jnp.float32),
        interpret=_INTERPRET,
    )(tile_expert, xg, gate_w, up_w, down_w)


def kernel(hidden_states, router_w, gate_w, up_w, down_w):
    B, S, D = hidden_states.shape
    T = B * S
    x = hidden_states.reshape(T, D)

    w, e = _route(x, router_w)

    e_flat = e.reshape(-1)
    oh = (e_flat[:, None] == jnp.arange(NUM_EXPERTS)[None, :]).astype(jnp.int32)
    pos = jnp.cumsum(oh, axis=0) - oh
    pos_i = jnp.sum(pos * oh, axis=-1)
    counts = jnp.sum(oh, axis=0)
    padded = ((counts + TM - 1) // TM) * TM
    ends = jnp.cumsum(padded)
    offs = ends - padded
    slot = offs[e_flat] + pos_i

    R = T * TOP_K + NUM_EXPERTS * TM
    gidx = jnp.zeros((R,), jnp.int32).at[slot].set(
        jnp.arange(T * TOP_K, dtype=jnp.int32) // TOP_K)
    n_tiles = R // TM
    tile_expert = jnp.minimum(
        jnp.sum(jnp.arange(n_tiles)[:, None] >= (ends // TM)[None, :], axis=-1),
        NUM_EXPERTS - 1).astype(jnp.int32)

    xg = jnp.take(x.astype(jnp.bfloat16), gidx, axis=0)
    y0 = _grouped_mlp_half(xg, tile_expert, gate_w, up_w, down_w, 0)
    y1 = _grouped_mlp_half(xg, tile_expert, gate_w, up_w, down_w, 1)

    s_tok = slot.reshape(T, TOP_K)
    out = (w[:, 0:1] * (jnp.take(y0, s_tok[:, 0], axis=0)
                        + jnp.take(y1, s_tok[:, 0], axis=0))
           + w[:, 1:2] * (jnp.take(y0, s_tok[:, 1], axis=0)
                          + jnp.take(y1, s_tok[:, 1], axis=0)))
    return out.reshape(B, S, D)
